# baseline (speedup 1.0000x reference)
import functools

import jax
import jax.numpy as jnp
from jax import lax
from jax.experimental import pallas as pl
from jax.experimental.pallas import tpu as pltpu
from jax.experimental.pallas import tpu_sc as plsc

BATCH = 4
SEQ = 2048
N_TOKENS = 8192
D = 8192

NC = 2
NS = 16
NW = NC * NS
B_TOTAL = BATCH * SEQ
BPW = B_TOTAL // NW
CH = 2
NBUF = 4
NCHUNK = BPW // CH
NG = NCHUNK // NBUF

_mesh = plsc.VectorSubcoreMesh(core_axis_name="c", subcore_axis_name="s")


@functools.partial(
    pl.kernel,
    mesh=_mesh,
    out_type=jax.ShapeDtypeStruct((B_TOTAL, D), jnp.float32),
    scratch_types=[
        pltpu.VMEM((NCHUNK, CH), jnp.int32),
        pltpu.VMEM((NBUF, CH, D), jnp.float32),
        pltpu.SemaphoreType.DMA,
        pltpu.SemaphoreType.DMA,
        pltpu.SemaphoreType.DMA,
        pltpu.SemaphoreType.DMA,
        pltpu.SemaphoreType.DMA,
        pltpu.SemaphoreType.DMA,
        pltpu.SemaphoreType.DMA,
        pltpu.SemaphoreType.DMA,
    ],
)
def _sc_gather(x_hbm, table_hbm, out_hbm, idx_v, rows_v,
               g0, g1, g2, g3, s0, s1, s2, s3):
    wid = lax.axis_index("s") * NC + lax.axis_index("c")
    base = wid * BPW
    gsems = (g0, g1, g2, g3)
    ssems = (s0, s1, s2, s3)
    pltpu.sync_copy(x_hbm.at[wid], idx_v)

    def g_start(slot, i):
        pltpu.async_copy(table_hbm.at[idx_v.at[i]], rows_v.at[slot],
                         gsems[slot])

    def g_wait(slot):
        pltpu.make_async_copy(table_hbm.at[pl.ds(0, CH)], rows_v.at[slot],
                              gsems[slot]).wait()

    def s_start(slot, i):
        pltpu.async_copy(rows_v.at[slot], out_hbm.at[pl.ds(base + i * CH, CH)],
                         ssems[slot])

    def s_wait(slot):
        pltpu.make_async_copy(rows_v.at[slot], out_hbm.at[pl.ds(0, CH)],
                              ssems[slot]).wait()

    def step(k, i, head=False, tail=False):
        ahead = (k + 2) % NBUF
        if not head:
            s_wait(ahead)
        if not tail:
            g_start(ahead, i + 2)
        g_wait(k)
        s_start(k, i)

    def group(g, head=False, tail=False):
        for k in range(NBUF):
            step(k, g * NBUF + k, head=head and k < 2, tail=tail and k >= 2)

    g_start(0, 0)
    g_start(1, 1)
    group(0, head=True)
    lax.fori_loop(1, NG - 1, lambda g, c: (group(g), c)[1], 0)
    group(NG - 1, tail=True)
    s_wait(2)
    s_wait(3)


def kernel(x, table):
    idx = x.reshape(NW, NCHUNK, CH).astype(jnp.int32)
    out = _sc_gather(idx, table)
    return out.reshape(BATCH, SEQ, D)


# speedup vs baseline: 1.0099x; 1.0099x over previous
import functools

import jax
import jax.numpy as jnp
from jax import lax
from jax.experimental import pallas as pl
from jax.experimental.pallas import tpu as pltpu
from jax.experimental.pallas import tpu_sc as plsc

BATCH = 4
SEQ = 2048
N_TOKENS = 8192
D = 8192

NC = 2
NS = 16
NW = NC * NS
B_TOTAL = BATCH * SEQ
BPW = B_TOTAL // NW
CH = 4
NBUF = 3
NCHUNK = BPW // CH

_mesh = plsc.VectorSubcoreMesh(core_axis_name="c", subcore_axis_name="s")


@functools.partial(
    pl.kernel,
    mesh=_mesh,
    out_type=jax.ShapeDtypeStruct((B_TOTAL, D), jnp.float32),
    scratch_types=[
        pltpu.VMEM((NCHUNK, CH), jnp.int32),
        pltpu.VMEM((NBUF, CH, D), jnp.float32),
        pltpu.SemaphoreType.DMA,
        pltpu.SemaphoreType.DMA,
        pltpu.SemaphoreType.DMA,
        pltpu.SemaphoreType.DMA,
        pltpu.SemaphoreType.DMA,
        pltpu.SemaphoreType.DMA,
    ],
)
def _sc_gather(x_hbm, table_hbm, out_hbm, idx_v, rows_v,
               g0, g1, g2, s0, s1, s2):
    wid = lax.axis_index("s") * NC + lax.axis_index("c")
    base = wid * BPW
    gsems = (g0, g1, g2)
    ssems = (s0, s1, s2)
    pltpu.sync_copy(x_hbm.at[wid], idx_v)

    def g_start(slot, i):
        pltpu.async_copy(table_hbm.at[idx_v.at[i]], rows_v.at[slot],
                         gsems[slot])

    def g_wait(slot):
        pltpu.make_async_copy(table_hbm.at[pl.ds(0, CH)], rows_v.at[slot],
                              gsems[slot]).wait()

    def s_start(slot, i):
        pltpu.async_copy(rows_v.at[slot], out_hbm.at[pl.ds(base + i * CH, CH)],
                         ssems[slot])

    def s_wait(slot):
        pltpu.make_async_copy(rows_v.at[slot], out_hbm.at[pl.ds(0, CH)],
                              ssems[slot]).wait()

    def step(k, i, drain=True, ahead=True):
        nxt = (k + 2) % NBUF
        if drain:
            s_wait(nxt)
        if ahead:
            g_start(nxt, i + 2)
        g_wait(k)
        s_start(k, i)

    g_start(0, 0)
    g_start(1, 1)
    step(0, 0, drain=False)
    step(1, 1)
    step(2, 2)
    lax.fori_loop(
        1, 20,
        lambda g, c: (step(0, 3 * g), step(1, 3 * g + 1), step(2, 3 * g + 2),
                      c)[-1],
        0)
    step(0, 60)
    step(1, 61)
    step(2, 62, ahead=False)
    step(0, 63, drain=False, ahead=False)
    s_wait(2)
    s_wait(0)


def kernel(x, table):
    idx = x.reshape(NW, NCHUNK, CH).astype(jnp.int32)
    out = _sc_gather(idx, table)
    return out.reshape(BATCH, SEQ, D)
